# baseline (speedup 1.0000x reference)
import jax
import jax.numpy as jnp
from jax.experimental import pallas as pl
from jax.experimental.pallas import tpu as pltpu

_B, _C, _L = 16, 128, 16384
_K, _S = 8, 4
_ITERS = 20
_W = (_L - _K) // _S + 1
_WP = 4096
_WSUB = 32
_NSUB = _WP // _WSUB
_LP = _S * _WP + _K


def _pool_kernel(q_ref, a_ref, x_ref, o_ref):
    alpha = jnp.exp(a_ref[...])
    q8 = _K * jax.nn.sigmoid(q_ref[...])
    inv_alpha = jnp.exp(-a_ref[...])

    def sub(j, _):
        base = j * (_S * _WSUB)
        xs = [alpha * x_ref[0, pl.ds(base + k, _WSUB, _S), :]
              for k in range(_K)]
        mn = xs[0]
        mx = xs[0]
        for t in xs[1:]:
            mn = jnp.minimum(mn, t)
            mx = jnp.maximum(mx, t)
        lo = mn - 2.0
        hi = mx + 2.0
        ps = [jnp.exp(t) for t in xs]
        for _i in range(_ITERS):
            c = 0.5 * (lo + hi)
            u = jnp.exp(c)
            acc = 1.0 / (u + ps[0])
            for p in ps[1:]:
                acc = acc + 1.0 / (u + p)
            s = u * acc
            th = s > q8
            lo = jnp.where(th, lo, c)
            hi = jnp.where(th, c, hi)
        o_ref[0, pl.ds(j * _WSUB, _WSUB), :] = (0.5 * (lo + hi)) * inv_alpha
        return 0

    jax.lax.fori_loop(0, _NSUB, sub, 0)


@jax.jit
def kernel(x, q_raw, alpha_raw):
    xt = jnp.transpose(x, (0, 2, 1))
    xt = jnp.pad(xt, ((0, 0), (0, _LP - _L), (0, 0)))
    q2 = q_raw.reshape(1, _C)
    a2 = alpha_raw.reshape(1, _C)
    out = pl.pallas_call(
        _pool_kernel,
        grid=(_B,),
        in_specs=[
            pl.BlockSpec((1, _C), lambda b: (0, 0)),
            pl.BlockSpec((1, _C), lambda b: (0, 0)),
            pl.BlockSpec((1, _LP, _C), lambda b: (b, 0, 0)),
        ],
        out_specs=pl.BlockSpec((1, _WP, _C), lambda b: (b, 0, 0)),
        out_shape=jax.ShapeDtypeStruct((_B, _WP, _C), jnp.float32),
        compiler_params=pltpu.CompilerParams(
            dimension_semantics=("parallel",),
            vmem_limit_bytes=48 * 1024 * 1024,
        ),
    )(q2, a2, xt)
    return out[:, :_W, :].transpose(0, 2, 1)


# speedup vs baseline: 1.1550x; 1.1550x over previous
import jax
import jax.numpy as jnp
from jax.experimental import pallas as pl
from jax.experimental.pallas import tpu as pltpu

_B, _C, _L = 16, 128, 16384
_K, _S = 8, 4
_ITERS = 20
_W = (_L - _K) // _S + 1
_WP = 4096
_WSUB = 32
_NSUB = _WP // _WSUB
_LP = _S * _WP + _K


def _pool_kernel(q_ref, a_ref, x_ref, o_ref):
    alpha = jnp.exp(a_ref[...])
    q8 = _K * jax.nn.sigmoid(q_ref[...])
    inv_alpha = jnp.exp(-a_ref[...])

    def one_chunk(j):
        base = j * (_S * _WSUB)
        xs = [alpha * x_ref[0, pl.ds(base + k, _WSUB, _S), :]
              for k in range(_K)]
        mn = xs[0]
        mx = xs[0]
        for t in xs[1:]:
            mn = jnp.minimum(mn, t)
            mx = jnp.maximum(mx, t)
        lo = mn - 2.0
        hi = mx + 2.0
        ps = [jnp.exp(t) for t in xs]
        for _i in range(_ITERS):
            c = 0.5 * (lo + hi)
            u = jnp.exp(c)
            acc = 1.0 / (u + ps[0])
            for p in ps[1:]:
                acc = acc + 1.0 / (u + p)
            s = u * acc
            th = s > q8
            lo = jnp.where(th, lo, c)
            hi = jnp.where(th, c, hi)
        o_ref[0, pl.ds(j * _WSUB, _WSUB), :] = (0.5 * (lo + hi)) * inv_alpha

    def sub(j, _):
        one_chunk(2 * j)
        one_chunk(2 * j + 1)
        return 0

    jax.lax.fori_loop(0, _NSUB // 2, sub, 0)


@jax.jit
def kernel(x, q_raw, alpha_raw):
    xt = jnp.transpose(x, (0, 2, 1))
    xt = jnp.pad(xt, ((0, 0), (0, _LP - _L), (0, 0)))
    q2 = q_raw.reshape(1, _C)
    a2 = alpha_raw.reshape(1, _C)
    out = pl.pallas_call(
        _pool_kernel,
        grid=(_B,),
        in_specs=[
            pl.BlockSpec((1, _C), lambda b: (0, 0)),
            pl.BlockSpec((1, _C), lambda b: (0, 0)),
            pl.BlockSpec((1, _LP, _C), lambda b: (b, 0, 0)),
        ],
        out_specs=pl.BlockSpec((1, _WP, _C), lambda b: (b, 0, 0)),
        out_shape=jax.ShapeDtypeStruct((_B, _WP, _C), jnp.float32),
        compiler_params=pltpu.CompilerParams(
            dimension_semantics=("arbitrary",),
            vmem_limit_bytes=48 * 1024 * 1024,
        ),
    )(q2, a2, xt)
    return out[:, :_W, :].transpose(0, 2, 1)


# speedup vs baseline: 1.2623x; 1.0929x over previous
import jax
import jax.numpy as jnp
from jax.experimental import pallas as pl
from jax.experimental.pallas import tpu as pltpu

_B, _C, _L = 16, 128, 16384
_K, _S = 8, 4
_ITERS = 20
_W = (_L - _K) // _S + 1
_WP = 4096
_WSUB = 32
_NSUB = _WP // _WSUB
_LP = _S * _WP + _K


def _pool_kernel(q_ref, a_ref, x_ref, o_ref):
    alpha = jnp.exp(a_ref[...])
    q8 = _K * jax.nn.sigmoid(q_ref[...])
    inv_alpha = jnp.exp(-a_ref[...])

    def one_chunk(j):
        base = j * (_S * _WSUB)
        xs = [alpha * x_ref[0, pl.ds(base + k, _WSUB, _S), :]
              for k in range(_K)]
        mn = xs[0]
        mx = xs[0]
        for t in xs[1:]:
            mn = jnp.minimum(mn, t)
            mx = jnp.maximum(mx, t)
        lo = mn - 2.0
        hi = mx + 2.0
        ps = [jnp.exp(t) for t in xs]
        for _i in range(_ITERS):
            c = 0.5 * (lo + hi)
            u = jnp.exp(c)
            acc = 1.0 / (u + ps[0])
            for p in ps[1:]:
                acc = acc + 1.0 / (u + p)
            s = u * acc
            th = s > q8
            lo = jnp.where(th, lo, c)
            hi = jnp.where(th, c, hi)
        o_ref[0, pl.ds(j * _WSUB, _WSUB), :] = (0.5 * (lo + hi)) * inv_alpha

    def sub(j, _):
        one_chunk(4 * j)
        one_chunk(4 * j + 1)
        one_chunk(4 * j + 2)
        one_chunk(4 * j + 3)
        return 0

    jax.lax.fori_loop(0, _NSUB // 4, sub, 0)


@jax.jit
def kernel(x, q_raw, alpha_raw):
    xt = jnp.transpose(x, (0, 2, 1))
    xt = jnp.pad(xt, ((0, 0), (0, _LP - _L), (0, 0)))
    q2 = q_raw.reshape(1, _C)
    a2 = alpha_raw.reshape(1, _C)
    out = pl.pallas_call(
        _pool_kernel,
        grid=(_B,),
        in_specs=[
            pl.BlockSpec((1, _C), lambda b: (0, 0)),
            pl.BlockSpec((1, _C), lambda b: (0, 0)),
            pl.BlockSpec((1, _LP, _C), lambda b: (b, 0, 0)),
        ],
        out_specs=pl.BlockSpec((1, _WP, _C), lambda b: (b, 0, 0)),
        out_shape=jax.ShapeDtypeStruct((_B, _WP, _C), jnp.float32),
        compiler_params=pltpu.CompilerParams(
            dimension_semantics=("arbitrary",),
            vmem_limit_bytes=48 * 1024 * 1024,
        ),
    )(q2, a2, xt)
    return out[:, :_W, :].transpose(0, 2, 1)


# speedup vs baseline: 1.3391x; 1.0608x over previous
import jax
import jax.numpy as jnp
from jax.experimental import pallas as pl
from jax.experimental.pallas import tpu as pltpu

_B, _C, _L = 16, 128, 16384
_K, _S = 8, 4
_ITERS = 20
_W = (_L - _K) // _S + 1
_WP = 4096
_WSUB = 32
_NSUB = _WP // _WSUB
_LP = _S * _WP + _K


def _pool_kernel(q_ref, a_ref, x_ref, o_ref):
    alpha = jnp.exp(a_ref[...])
    q8 = _K * jax.nn.sigmoid(q_ref[...])
    inv_alpha = jnp.exp(-a_ref[...])

    def one_chunk(j):
        base = j * (_S * _WSUB)
        xs = [alpha * x_ref[0, pl.ds(base + k, _WSUB, _S), :]
              for k in range(_K)]
        mn = xs[0]
        mx = xs[0]
        for t in xs[1:]:
            mn = jnp.minimum(mn, t)
            mx = jnp.maximum(mx, t)
        lo = mn - 2.0
        hi = mx + 2.0
        ps = [jnp.exp(t) for t in xs]
        for _i in range(_ITERS):
            c = 0.5 * (lo + hi)
            u = jnp.exp(c)
            acc = 1.0 / (u + ps[0])
            for p in ps[1:]:
                acc = acc + 1.0 / (u + p)
            s = u * acc
            th = s > q8
            lo = jnp.where(th, lo, c)
            hi = jnp.where(th, c, hi)
        o_ref[0, pl.ds(j * _WSUB, _WSUB), :] = (0.5 * (lo + hi)) * inv_alpha

    def sub(j, _):
        for g in range(8):
            one_chunk(8 * j + g)
        return 0

    jax.lax.fori_loop(0, _NSUB // 8, sub, 0)


@jax.jit
def kernel(x, q_raw, alpha_raw):
    xt = jnp.transpose(x, (0, 2, 1))
    xt = jnp.pad(xt, ((0, 0), (0, _LP - _L), (0, 0)))
    q2 = q_raw.reshape(1, _C)
    a2 = alpha_raw.reshape(1, _C)
    out = pl.pallas_call(
        _pool_kernel,
        grid=(_B,),
        in_specs=[
            pl.BlockSpec((1, _C), lambda b: (0, 0)),
            pl.BlockSpec((1, _C), lambda b: (0, 0)),
            pl.BlockSpec((1, _LP, _C), lambda b: (b, 0, 0)),
        ],
        out_specs=pl.BlockSpec((1, _WP, _C), lambda b: (b, 0, 0)),
        out_shape=jax.ShapeDtypeStruct((_B, _WP, _C), jnp.float32),
        compiler_params=pltpu.CompilerParams(
            dimension_semantics=("arbitrary",),
            vmem_limit_bytes=48 * 1024 * 1024,
        ),
    )(q2, a2, xt)
    return out[:, :_W, :].transpose(0, 2, 1)


# speedup vs baseline: 1.5564x; 1.1622x over previous
import jax
import jax.numpy as jnp
from jax.experimental import pallas as pl
from jax.experimental.pallas import tpu as pltpu

_B, _C, _L = 16, 128, 16384
_K, _S = 8, 4
_ITERS = 20
_W = (_L - _K) // _S + 1
_WP = 4096
_WSUB = 32
_NSUB = _WP // _WSUB
_GRP = 8
_LP = _S * _WP + _K


def _pool_kernel(q_ref, a_ref, x_ref, o_ref):
    half_alpha = 0.5 * jnp.exp(a_ref[...])
    t8 = _K * (2.0 * jax.nn.sigmoid(q_ref[...]) - 1.0)
    inv_alpha = jnp.exp(-a_ref[...])

    def one_chunk(j):
        base = j * (_S * _WSUB)
        vs = [half_alpha * x_ref[0, pl.ds(base + k, _WSUB, _S), :]
              for k in range(_K)]
        mn = vs[0]
        mx = vs[0]
        for t in vs[1:]:
            mn = jnp.minimum(mn, t)
            mx = jnp.maximum(mx, t)
        lo = mn - 1.0
        hi = mx + 1.0
        for _i in range(_ITERS):
            y = 0.5 * (lo + hi)
            acc = jnp.tanh(y - vs[0])
            for v in vs[1:]:
                acc = acc + jnp.tanh(y - v)
            th = acc > t8
            lo = jnp.where(th, lo, y)
            hi = jnp.where(th, y, hi)
        o_ref[0, pl.ds(j * _WSUB, _WSUB), :] = (lo + hi) * inv_alpha

    def sub(j, _):
        for g in range(_GRP):
            one_chunk(_GRP * j + g)
        return 0

    jax.lax.fori_loop(0, _NSUB // _GRP, sub, 0)


@jax.jit
def kernel(x, q_raw, alpha_raw):
    xt = jnp.transpose(x, (0, 2, 1))
    xt = jnp.pad(xt, ((0, 0), (0, _LP - _L), (0, 0)))
    q2 = q_raw.reshape(1, _C)
    a2 = alpha_raw.reshape(1, _C)
    out = pl.pallas_call(
        _pool_kernel,
        grid=(_B,),
        in_specs=[
            pl.BlockSpec((1, _C), lambda b: (0, 0)),
            pl.BlockSpec((1, _C), lambda b: (0, 0)),
            pl.BlockSpec((1, _LP, _C), lambda b: (b, 0, 0)),
        ],
        out_specs=pl.BlockSpec((1, _WP, _C), lambda b: (b, 0, 0)),
        out_shape=jax.ShapeDtypeStruct((_B, _WP, _C), jnp.float32),
        compiler_params=pltpu.CompilerParams(
            dimension_semantics=("arbitrary",),
            vmem_limit_bytes=48 * 1024 * 1024,
        ),
    )(q2, a2, xt)
    return out[:, :_W, :].transpose(0, 2, 1)


# speedup vs baseline: 1.5625x; 1.0040x over previous
import jax
import jax.numpy as jnp
from jax.experimental import pallas as pl
from jax.experimental.pallas import tpu as pltpu

_B, _C, _L = 16, 128, 16384
_K, _S = 8, 4
_ITERS = 20
_W = (_L - _K) // _S + 1
_WP = 4096
_WSUB = 32
_NSUB = _WP // _WSUB
_GRP = 8
_LP = _S * _WP + _K


def _pool_kernel(q_ref, a_ref, x_ref, o_ref):
    half_alpha = 0.5 * jnp.exp(a_ref[...])
    t8 = _K * (2.0 * jax.nn.sigmoid(q_ref[...]) - 1.0)
    two_inv_alpha = 2.0 * jnp.exp(-a_ref[...])

    def one_chunk(j):
        base = j * (_S * _WSUB)
        vs = [half_alpha * x_ref[0, pl.ds(base + k, _WSUB, _S), :]
              for k in range(_K)]
        mn = vs[0]
        mx = vs[0]
        for t in vs[1:]:
            mn = jnp.minimum(mn, t)
            mx = jnp.maximum(mx, t)
        y = 0.5 * (mn + mx)
        d = 0.25 * (mx - mn) + 0.5
        for _i in range(_ITERS):
            acc = jnp.tanh(y - vs[0])
            for v in vs[1:]:
                acc = acc + jnp.tanh(y - v)
            th = acc > t8
            y = y + jnp.where(th, -d, d)
            d = 0.5 * d
        o_ref[0, pl.ds(j * _WSUB, _WSUB), :] = y * two_inv_alpha

    def sub(j, _):
        for g in range(_GRP):
            one_chunk(_GRP * j + g)
        return 0

    jax.lax.fori_loop(0, _NSUB // _GRP, sub, 0)


@jax.jit
def kernel(x, q_raw, alpha_raw):
    xt = jnp.transpose(x, (0, 2, 1))
    xt = jnp.pad(xt, ((0, 0), (0, _LP - _L), (0, 0)))
    q2 = q_raw.reshape(1, _C)
    a2 = alpha_raw.reshape(1, _C)
    out = pl.pallas_call(
        _pool_kernel,
        grid=(_B,),
        in_specs=[
            pl.BlockSpec((1, _C), lambda b: (0, 0)),
            pl.BlockSpec((1, _C), lambda b: (0, 0)),
            pl.BlockSpec((1, _LP, _C), lambda b: (b, 0, 0)),
        ],
        out_specs=pl.BlockSpec((1, _WP, _C), lambda b: (b, 0, 0)),
        out_shape=jax.ShapeDtypeStruct((_B, _WP, _C), jnp.float32),
        compiler_params=pltpu.CompilerParams(
            dimension_semantics=("arbitrary",),
            vmem_limit_bytes=48 * 1024 * 1024,
        ),
    )(q2, a2, xt)
    return out[:, :_W, :].transpose(0, 2, 1)


# speedup vs baseline: 1.7534x; 1.1221x over previous
import jax
import jax.numpy as jnp
from jax.experimental import pallas as pl
from jax.experimental.pallas import tpu as pltpu

_B, _C, _L = 16, 128, 16384
_K, _S = 8, 4
_ITERS = 20
_W = (_L - _K) // _S + 1
_WP = 4096
_WSUB = 32
_GRP = 8
_WG = _WSUB * _GRP
_NG = _WP // _WG
_NT = (_S * _WG) // _C + 1


def _pool_kernel(q_ref, a_ref, x_ref, o_ref, s_ref):
    g = pl.program_id(1)
    half_alpha = 0.5 * jnp.exp(a_ref[...])
    t8 = _K * (2.0 * jax.nn.sigmoid(q_ref[...]) - 1.0)
    two_inv_alpha = 2.0 * jnp.exp(-a_ref[...])

    for t in range(_NT - 1):
        lane0 = pl.multiple_of(g * (_S * _WG) + t * _C, _C)
        s_ref[pl.ds(t * _C, _C), :] = jnp.transpose(
            x_ref[0, :, pl.ds(lane0, _C)])

    t_last = _NT - 1

    @pl.when(g < _NG - 1)
    def _():
        lane0 = pl.multiple_of(g * (_S * _WG) + t_last * _C, _C)
        s_ref[pl.ds(t_last * _C, _C), :] = jnp.transpose(
            x_ref[0, :, pl.ds(lane0, _C)])

    @pl.when(g == _NG - 1)
    def _():
        s_ref[pl.ds(t_last * _C, 8), :] = jnp.zeros((8, _C), jnp.float32)

    def one_chunk(j):
        base = j * (_S * _WSUB)
        vs = [half_alpha * s_ref[pl.ds(base + k, _WSUB, _S), :]
              for k in range(_K)]
        mn = vs[0]
        mx = vs[0]
        for t in vs[1:]:
            mn = jnp.minimum(mn, t)
            mx = jnp.maximum(mx, t)
        y = 0.5 * (mn + mx)
        d = 0.25 * (mx - mn) + 0.5
        for _i in range(_ITERS):
            acc = jnp.tanh(y - vs[0])
            for v in vs[1:]:
                acc = acc + jnp.tanh(y - v)
            th = acc > t8
            y = y + jnp.where(th, -d, d)
            d = 0.5 * d
        return y * two_inv_alpha

    res = [one_chunk(j) for j in range(_GRP)]
    for h in range(2):
        blk = jnp.concatenate(res[4 * h:4 * h + 4], axis=0)
        o_ref[0, :, pl.ds(h * _C, _C)] = jnp.transpose(blk)


@jax.jit
def kernel(x, q_raw, alpha_raw):
    q2 = q_raw.reshape(1, _C)
    a2 = alpha_raw.reshape(1, _C)
    out = pl.pallas_call(
        _pool_kernel,
        grid=(_B, _NG),
        in_specs=[
            pl.BlockSpec((1, _C), lambda b, g: (0, 0)),
            pl.BlockSpec((1, _C), lambda b, g: (0, 0)),
            pl.BlockSpec((1, _C, _L), lambda b, g: (b, 0, 0)),
        ],
        out_specs=pl.BlockSpec((1, _C, _WG), lambda b, g: (b, 0, g)),
        out_shape=jax.ShapeDtypeStruct((_B, _C, _WP), jnp.float32),
        scratch_shapes=[pltpu.VMEM((_NT * _C, _C), jnp.float32)],
        compiler_params=pltpu.CompilerParams(
            dimension_semantics=("arbitrary", "arbitrary"),
            vmem_limit_bytes=48 * 1024 * 1024,
        ),
    )(q2, a2, x)
    return out[:, :, :_W]


# speedup vs baseline: 1.8070x; 1.0306x over previous
import jax
import jax.numpy as jnp
from jax.experimental import pallas as pl
from jax.experimental.pallas import tpu as pltpu

_B, _C, _L = 16, 128, 16384
_K, _S = 8, 4
_ITERS = 20
_W = (_L - _K) // _S + 1
_WSUB = 32
_GRP = 8
_WG = _WSUB * _GRP
_NG = 16
_NTILES = _L // _C


def _pool_kernel(q_ref, a_ref, x_ref, o_ref, s_ref):
    g = pl.program_id(1)
    half_alpha = 0.5 * jnp.exp(a_ref[...])
    t8 = _K * (2.0 * jax.nn.sigmoid(q_ref[...]) - 1.0)
    two_inv_alpha = 2.0 * jnp.exp(-a_ref[...])

    @pl.when(g == 0)
    def _():
        for t in range(_NTILES):
            s_ref[pl.ds(t * _C, _C), :] = jnp.transpose(
                x_ref[0, :, pl.ds(t * _C, _C)])
        s_ref[pl.ds(_L, 8), :] = jnp.zeros((8, _C), jnp.float32)

    def one_chunk(j):
        base = g * (_S * _WG) + j * (_S * _WSUB)
        vs = [half_alpha * s_ref[pl.ds(base + k, _WSUB, _S), :]
              for k in range(_K)]
        mn = vs[0]
        mx = vs[0]
        for t in vs[1:]:
            mn = jnp.minimum(mn, t)
            mx = jnp.maximum(mx, t)
        y = 0.5 * (mn + mx)
        d = 0.25 * (mx - mn) + 0.5
        for _i in range(_ITERS):
            acc = jnp.tanh(y - vs[0])
            for v in vs[1:]:
                acc = acc + jnp.tanh(y - v)
            th = acc > t8
            y = y + jnp.where(th, -d, d)
            d = 0.5 * d
        return y * two_inv_alpha

    res = [one_chunk(j) for j in range(_GRP)]
    for h in range(2):
        blk = jnp.concatenate(res[4 * h:4 * h + 4], axis=0)
        o_ref[0, :, pl.ds(h * _C, _C)] = jnp.transpose(blk)


@jax.jit
def kernel(x, q_raw, alpha_raw):
    q2 = q_raw.reshape(1, _C)
    a2 = alpha_raw.reshape(1, _C)
    out = pl.pallas_call(
        _pool_kernel,
        grid=(_B, _NG),
        in_specs=[
            pl.BlockSpec((1, _C), lambda b, g: (0, 0)),
            pl.BlockSpec((1, _C), lambda b, g: (0, 0)),
            pl.BlockSpec((1, _C, _L), lambda b, g: (b, 0, 0)),
        ],
        out_specs=pl.BlockSpec((1, _C, _WG), lambda b, g: (b, 0, g)),
        out_shape=jax.ShapeDtypeStruct((_B, _C, _W), jnp.float32),
        scratch_shapes=[pltpu.VMEM((_L + 8, _C), jnp.float32)],
        compiler_params=pltpu.CompilerParams(
            dimension_semantics=("arbitrary", "arbitrary"),
            vmem_limit_bytes=48 * 1024 * 1024,
        ),
    )(q2, a2, x)
    return out
